# Initial kernel scaffold; baseline (speedup 1.0000x reference)
import jax
import jax.numpy as jnp
from jax.experimental import pallas as pl


def kernel(x, edge_index, W1, b1, W2, b2):
    raise NotImplementedError("write your pallas kernel here")


# speedup vs baseline: 13.3546x; 13.3546x over previous
import functools

import jax
import jax.numpy as jnp
from jax import lax
from jax.experimental import pallas as pl
from jax.experimental.pallas import tpu as pltpu
from jax.experimental.pallas import tpu_sc as plsc

N = 10000
E = 320000
F_IN = 128
H = 64
C = 40
C_PAD = 48

NC = 2
NS = 16
NW = NC * NS
N_PAD = 10240
RPT = N_PAD // NS
EPW = E // NW
CHUNK = 80
NCHUNK = EPW // CHUNK

_MESH = dict(core_axis_name="c", subcore_axis_name="s",
             num_cores=NC, num_subcores=NS)


def _worker_id():
    return lax.axis_index("s") * NC + lax.axis_index("c")


def _sc_degree(dst3, zeros_rows):
    mesh = plsc.VectorSubcoreMesh(**_MESH)

    @functools.partial(
        pl.kernel,
        out_type=jax.ShapeDtypeStruct((NC, NS, RPT), jnp.float32),
        mesh=mesh,
        scratch_types=[
            pltpu.VMEM((CHUNK,), jnp.int32),
            pltpu.VMEM((CHUNK,), jnp.float32),
            pltpu.VMEM_SHARED((N_PAD,), jnp.float32),
        ],
    )
    def k(dst_hbm, z_hbm, deg_out, idx_v, ones_v, acc):
        cid = lax.axis_index("c")
        sid = lax.axis_index("s")
        wid = _worker_id()
        for i in range(CHUNK // 16):
            ones_v[pl.ds(16 * i, 16)] = jnp.full((16,), 1.0, jnp.float32)
        pltpu.sync_copy(z_hbm.at[sid], acc.at[pl.ds(sid * RPT, RPT)])
        plsc.subcore_barrier()

        def body(ci, carry):
            pltpu.sync_copy(dst_hbm.at[wid, ci], idx_v)
            pltpu.sync_copy(ones_v, acc.at[idx_v], add=True)
            return carry

        lax.fori_loop(0, NCHUNK, body, 0)
        plsc.subcore_barrier()
        pltpu.sync_copy(acc.at[pl.ds(sid * RPT, RPT)], deg_out.at[cid, sid])

    return k(dst3, zeros_rows)


def _sc_scatter_rows(g, src3, dst3, zeros_rows, d):
    mesh = plsc.VectorSubcoreMesh(**_MESH)

    @functools.partial(
        pl.kernel,
        out_type=jax.ShapeDtypeStruct((NC, NS, RPT, d), jnp.float32),
        mesh=mesh,
        scratch_types=[
            pltpu.VMEM((CHUNK,), jnp.int32),
            pltpu.VMEM((CHUNK,), jnp.int32),
            pltpu.VMEM((CHUNK, d), jnp.float32),
            pltpu.VMEM_SHARED((N_PAD, d), jnp.float32),
            pltpu.SemaphoreType.DMA,
        ],
        compiler_params=pltpu.CompilerParams(use_tc_tiling_on_sc=False),
    )
    def k(g_hbm, src_hbm, dst_hbm, z_hbm, out, idx_g, idx_s, rows, acc, sem):
        cid = lax.axis_index("c")
        sid = lax.axis_index("s")
        wid = _worker_id()
        pltpu.sync_copy(z_hbm.at[sid], acc.at[pl.ds(sid * RPT, RPT)])
        plsc.subcore_barrier()

        def body(ci, carry):
            pltpu.sync_copy(src_hbm.at[wid, ci], idx_g)
            pltpu.async_copy(g_hbm.at[idx_g], rows, sem).wait()
            pltpu.sync_copy(dst_hbm.at[wid, ci], idx_s)
            pltpu.sync_copy(rows, acc.at[idx_s], add=True)
            return carry

        lax.fori_loop(0, NCHUNK, body, 0)
        plsc.subcore_barrier()
        pltpu.sync_copy(acc.at[pl.ds(sid * RPT, RPT)], out.at[cid, sid])

    return k(g, src3, dst3, zeros_rows)


BM = 320


def _dis_from(degt):
    deg = degt[:, 0:1] + degt[:, 1:2] + 1.0
    return lax.rsqrt(deg)


def _tc_g1(x, w1, degt):
    def body(x_ref, w_ref, deg_ref, o_ref):
        dis = _dis_from(deg_ref[...])
        h = jnp.dot(x_ref[...], w_ref[...], preferred_element_type=jnp.float32)
        o_ref[...] = h * dis

    return pl.pallas_call(
        body,
        grid=(N_PAD // BM,),
        in_specs=[
            pl.BlockSpec((BM, F_IN), lambda i: (i, 0)),
            pl.BlockSpec((F_IN, H), lambda i: (0, 0)),
            pl.BlockSpec((BM, 2), lambda i: (i, 0)),
        ],
        out_specs=pl.BlockSpec((BM, H), lambda i: (i, 0)),
        out_shape=jax.ShapeDtypeStruct((N_PAD, H), jnp.float32),
    )(x, w1, degt)


def _tc_g2(g1, p0, p1, degt, b1, w2p):
    def body(g_ref, p0_ref, p1_ref, deg_ref, b_ref, w_ref, o_ref):
        dis = _dis_from(deg_ref[...])
        z = dis * (p0_ref[...] + p1_ref[...] + g_ref[...]) + b_ref[...]
        z = jnp.maximum(z, 0.0)
        o_ref[...] = jnp.dot(z, w_ref[...], preferred_element_type=jnp.float32) * dis

    return pl.pallas_call(
        body,
        grid=(N_PAD // BM,),
        in_specs=[
            pl.BlockSpec((BM, H), lambda i: (i, 0)),
            pl.BlockSpec((BM, H), lambda i: (i, 0)),
            pl.BlockSpec((BM, H), lambda i: (i, 0)),
            pl.BlockSpec((BM, 2), lambda i: (i, 0)),
            pl.BlockSpec((1, H), lambda i: (0, 0)),
            pl.BlockSpec((H, C_PAD), lambda i: (0, 0)),
        ],
        out_specs=pl.BlockSpec((BM, C_PAD), lambda i: (i, 0)),
        out_shape=jax.ShapeDtypeStruct((N_PAD, C_PAD), jnp.float32),
    )(g1, p0, p1, degt, b1, w2p)


def _tc_out(g2, q0, q1, degt, b2p):
    def body(g_ref, q0_ref, q1_ref, deg_ref, b_ref, o_ref):
        dis = _dis_from(deg_ref[...])
        res = dis * (q0_ref[...] + q1_ref[...] + g_ref[...]) + b_ref[...]
        o_ref[...] = res[:, :C]

    return pl.pallas_call(
        body,
        grid=(N_PAD // BM,),
        in_specs=[
            pl.BlockSpec((BM, C_PAD), lambda i: (i, 0)),
            pl.BlockSpec((BM, C_PAD), lambda i: (i, 0)),
            pl.BlockSpec((BM, C_PAD), lambda i: (i, 0)),
            pl.BlockSpec((BM, 2), lambda i: (i, 0)),
            pl.BlockSpec((1, C_PAD), lambda i: (0, 0)),
        ],
        out_specs=pl.BlockSpec((BM, C), lambda i: (i, 0)),
        out_shape=jax.ShapeDtypeStruct((N_PAD, C), jnp.float32),
    )(g2, q0, q1, degt, b2p)


def kernel(x, edge_index, W1, b1, W2, b2):
    src3 = edge_index[0].reshape(NW, NCHUNK, CHUNK)
    dst3 = edge_index[1].reshape(NW, NCHUNK, CHUNK)
    x_pad = jnp.pad(x, ((0, N_PAD - N), (0, 0)))
    w2p = jnp.pad(W2, ((0, 0), (0, C_PAD - C)))
    b1r = b1.reshape(1, H)
    b2p = jnp.pad(b2, (0, C_PAD - C)).reshape(1, C_PAD)
    z1 = jnp.zeros((NS, RPT), jnp.float32)
    zh = jnp.zeros((NS, RPT, H), jnp.float32)
    zc = jnp.zeros((NS, RPT, C_PAD), jnp.float32)

    degp = _sc_degree(dst3, z1)
    degt = degp.reshape(NC, N_PAD).T

    g1 = _tc_g1(x_pad, W1, degt)
    p = _sc_scatter_rows(g1, src3, dst3, zh, H)
    p = p.reshape(NC, N_PAD, H)
    g2 = _tc_g2(g1, p[0], p[1], degt, b1r, w2p)
    q = _sc_scatter_rows(g2, src3, dst3, zc, C_PAD)
    q = q.reshape(NC, N_PAD, C_PAD)
    out = _tc_out(g2, q[0], q[1], degt, b2p)
    return out[:N]


# speedup vs baseline: 27.0655x; 2.0267x over previous
import functools

import jax
import jax.numpy as jnp
from jax import lax
from jax.experimental import pallas as pl
from jax.experimental.pallas import tpu as pltpu
from jax.experimental.pallas import tpu_sc as plsc

N = 10000
E = 320000
F_IN = 128
H = 64
C = 40
C_PAD = 48

NC = 2
NS = 16
NW = NC * NS
N_PAD = 10240
RPT = N_PAD // NS
EPW = E // NW
CHUNK = 80
NCHUNK = EPW // CHUNK

_MESH = dict(core_axis_name="c", subcore_axis_name="s",
             num_cores=NC, num_subcores=NS)


def _worker_id():
    return lax.axis_index("s") * NC + lax.axis_index("c")


def _sc_degree(dst3, zeros_rows):
    mesh = plsc.VectorSubcoreMesh(**_MESH)

    @functools.partial(
        pl.kernel,
        out_type=jax.ShapeDtypeStruct((NC, NS, RPT), jnp.float32),
        mesh=mesh,
        scratch_types=[
            pltpu.VMEM((NCHUNK, CHUNK), jnp.int32),
            pltpu.VMEM((CHUNK,), jnp.float32),
            pltpu.VMEM_SHARED((N_PAD,), jnp.float32),
        ],
    )
    def k(dst_hbm, z_hbm, deg_out, eidx, ones_v, acc):
        cid = lax.axis_index("c")
        sid = lax.axis_index("s")
        wid = _worker_id()
        for i in range(CHUNK // 16):
            ones_v[pl.ds(16 * i, 16)] = jnp.full((16,), 1.0, jnp.float32)
        pltpu.sync_copy(dst_hbm.at[wid], eidx)
        pltpu.sync_copy(z_hbm.at[sid], acc.at[pl.ds(sid * RPT, RPT)])
        plsc.subcore_barrier()

        def body(ci, carry):
            pltpu.sync_copy(ones_v, acc.at[eidx.at[ci]], add=True)
            return carry

        lax.fori_loop(0, NCHUNK, body, 0)
        plsc.subcore_barrier()
        pltpu.sync_copy(acc.at[pl.ds(sid * RPT, RPT)], deg_out.at[cid, sid])

    return k(dst3, zeros_rows)


def _sc_scatter_rows(g, edges4, zeros_rows, d):
    mesh = plsc.VectorSubcoreMesh(**_MESH)

    @functools.partial(
        pl.kernel,
        out_type=jax.ShapeDtypeStruct((NC, NS, RPT, d), jnp.float32),
        mesh=mesh,
        scratch_types=[
            pltpu.VMEM((NCHUNK, 2, CHUNK), jnp.int32),
            pltpu.VMEM((CHUNK, d), jnp.float32),
            pltpu.VMEM((CHUNK, d), jnp.float32),
            pltpu.VMEM_SHARED((N_PAD, d), jnp.float32),
            pltpu.SemaphoreType.DMA,
            pltpu.SemaphoreType.DMA,
        ],
        compiler_params=pltpu.CompilerParams(use_tc_tiling_on_sc=False),
    )
    def k(g_hbm, e_hbm, z_hbm, out, eidx, rows_a, rows_b, acc, sem_a, sem_b):
        cid = lax.axis_index("c")
        sid = lax.axis_index("s")
        wid = _worker_id()
        pltpu.sync_copy(e_hbm.at[wid], eidx)
        pltpu.sync_copy(z_hbm.at[sid], acc.at[pl.ds(sid * RPT, RPT)])
        plsc.subcore_barrier()

        def gather_start(ci, rows, sem):
            pltpu.async_copy(g_hbm.at[eidx.at[ci, 0]], rows, sem)

        def gather_wait(ci, rows, sem):
            pltpu.make_async_copy(g_hbm.at[eidx.at[ci, 0]], rows, sem).wait()

        def scatter(ci, rows):
            pltpu.sync_copy(rows, acc.at[eidx.at[ci, 1]], add=True)

        gather_start(0, rows_a, sem_a)

        def body(i, carry):
            a = 2 * i
            b = a + 1
            gather_start(b, rows_b, sem_b)
            gather_wait(a, rows_a, sem_a)
            scatter(a, rows_a)
            gather_start(a + 2, rows_a, sem_a)
            gather_wait(b, rows_b, sem_b)
            scatter(b, rows_b)
            return carry

        lax.fori_loop(0, (NCHUNK - 1) // 2, body, 0)
        last = NCHUNK - 1
        gather_wait(last, rows_a, sem_a)
        scatter(last, rows_a)
        plsc.subcore_barrier()
        pltpu.sync_copy(acc.at[pl.ds(sid * RPT, RPT)], out.at[cid, sid])

    return k(g, edges4, zeros_rows)


BM = 320


def _dis_from(degt):
    deg = degt[:, 0:1] + degt[:, 1:2] + 1.0
    return lax.rsqrt(deg)


def _tc_g1(x, w1, degt):
    def body(x_ref, w_ref, deg_ref, o_ref):
        dis = _dis_from(deg_ref[...])
        h = jnp.dot(x_ref[...], w_ref[...], preferred_element_type=jnp.float32)
        o_ref[...] = h * dis

    return pl.pallas_call(
        body,
        grid=(N_PAD // BM,),
        in_specs=[
            pl.BlockSpec((BM, F_IN), lambda i: (i, 0)),
            pl.BlockSpec((F_IN, H), lambda i: (0, 0)),
            pl.BlockSpec((BM, 2), lambda i: (i, 0)),
        ],
        out_specs=pl.BlockSpec((BM, H), lambda i: (i, 0)),
        out_shape=jax.ShapeDtypeStruct((N_PAD, H), jnp.float32),
    )(x, w1, degt)


def _tc_g2(g1, p0, p1, degt, b1, w2p):
    def body(g_ref, p0_ref, p1_ref, deg_ref, b_ref, w_ref, o_ref):
        dis = _dis_from(deg_ref[...])
        z = dis * (p0_ref[...] + p1_ref[...] + g_ref[...]) + b_ref[...]
        z = jnp.maximum(z, 0.0)
        o_ref[...] = jnp.dot(z, w_ref[...], preferred_element_type=jnp.float32) * dis

    return pl.pallas_call(
        body,
        grid=(N_PAD // BM,),
        in_specs=[
            pl.BlockSpec((BM, H), lambda i: (i, 0)),
            pl.BlockSpec((BM, H), lambda i: (i, 0)),
            pl.BlockSpec((BM, H), lambda i: (i, 0)),
            pl.BlockSpec((BM, 2), lambda i: (i, 0)),
            pl.BlockSpec((1, H), lambda i: (0, 0)),
            pl.BlockSpec((H, C_PAD), lambda i: (0, 0)),
        ],
        out_specs=pl.BlockSpec((BM, C_PAD), lambda i: (i, 0)),
        out_shape=jax.ShapeDtypeStruct((N_PAD, C_PAD), jnp.float32),
    )(g1, p0, p1, degt, b1, w2p)


def _tc_out(g2, q0, q1, degt, b2p):
    def body(g_ref, q0_ref, q1_ref, deg_ref, b_ref, o_ref):
        dis = _dis_from(deg_ref[...])
        res = dis * (q0_ref[...] + q1_ref[...] + g_ref[...]) + b_ref[...]
        o_ref[...] = res[:, :C]

    return pl.pallas_call(
        body,
        grid=(N_PAD // BM,),
        in_specs=[
            pl.BlockSpec((BM, C_PAD), lambda i: (i, 0)),
            pl.BlockSpec((BM, C_PAD), lambda i: (i, 0)),
            pl.BlockSpec((BM, C_PAD), lambda i: (i, 0)),
            pl.BlockSpec((BM, 2), lambda i: (i, 0)),
            pl.BlockSpec((1, C_PAD), lambda i: (0, 0)),
        ],
        out_specs=pl.BlockSpec((BM, C), lambda i: (i, 0)),
        out_shape=jax.ShapeDtypeStruct((N_PAD, C), jnp.float32),
    )(g2, q0, q1, degt, b2p)


def kernel(x, edge_index, W1, b1, W2, b2):
    src3 = edge_index[0].reshape(NW, NCHUNK, CHUNK)
    dst3 = edge_index[1].reshape(NW, NCHUNK, CHUNK)
    edges4 = jnp.stack([src3, dst3], axis=2)
    x_pad = jnp.pad(x, ((0, N_PAD - N), (0, 0)))
    w2p = jnp.pad(W2, ((0, 0), (0, C_PAD - C)))
    b1r = b1.reshape(1, H)
    b2p = jnp.pad(b2, (0, C_PAD - C)).reshape(1, C_PAD)
    z1 = jnp.zeros((NS, RPT), jnp.float32)
    zh = jnp.zeros((NS, RPT, H), jnp.float32)
    zc = jnp.zeros((NS, RPT, C_PAD), jnp.float32)

    degp = _sc_degree(dst3, z1)
    degt = degp.reshape(NC, N_PAD).T

    g1 = _tc_g1(x_pad, W1, degt)
    p = _sc_scatter_rows(g1, edges4, zh, H)
    p = p.reshape(NC, N_PAD, H)
    g2 = _tc_g2(g1, p[0], p[1], degt, b1r, w2p)
    q = _sc_scatter_rows(g2, edges4, zc, C_PAD)
    q = q.reshape(NC, N_PAD, C_PAD)
    out = _tc_out(g2, q[0], q[1], degt, b2p)
    return out[:N]


# speedup vs baseline: 35.4363x; 1.3093x over previous
import functools

import jax
import jax.numpy as jnp
from jax import lax
from jax.experimental import pallas as pl
from jax.experimental.pallas import tpu as pltpu
from jax.experimental.pallas import tpu_sc as plsc

N = 10000
E = 320000
F_IN = 128
H = 64
C = 40
C_PAD = 48

NC = 2
NS = 16
NW = NC * NS
N_PAD = 10240
RPT = N_PAD // NS
EPW = E // NW
CHUNK = 80
NCHUNK = EPW // CHUNK

_MESH = dict(core_axis_name="c", subcore_axis_name="s",
             num_cores=NC, num_subcores=NS)


def _worker_id():
    return lax.axis_index("s") * NC + lax.axis_index("c")


def _sc_degree(dst3, zeros_rows):
    mesh = plsc.VectorSubcoreMesh(**_MESH)

    @functools.partial(
        pl.kernel,
        out_type=(jax.ShapeDtypeStruct((N_PAD,), jnp.float32),
                  jax.ShapeDtypeStruct((N_PAD,), jnp.float32)),
        mesh=mesh,
        scratch_types=[
            pltpu.VMEM((NCHUNK, CHUNK), jnp.int32),
            pltpu.VMEM((CHUNK,), jnp.float32),
            pltpu.VMEM_SHARED((N_PAD,), jnp.float32),
        ],
    )
    def k(dst_hbm, z_hbm, out0, out1, eidx, ones_v, acc):
        cid = lax.axis_index("c")
        sid = lax.axis_index("s")
        wid = _worker_id()
        for i in range(CHUNK // 16):
            ones_v[pl.ds(16 * i, 16)] = jnp.full((16,), 1.0, jnp.float32)
        pltpu.sync_copy(dst_hbm.at[wid], eidx)
        pltpu.sync_copy(z_hbm.at[sid], acc.at[pl.ds(sid * RPT, RPT)])
        plsc.subcore_barrier()

        def body(ci, carry):
            pltpu.sync_copy(ones_v, acc.at[eidx.at[ci]], add=True)
            return carry

        lax.fori_loop(0, NCHUNK, body, 0)
        plsc.subcore_barrier()
        sl = pl.ds(sid * RPT, RPT)

        @pl.when(cid == 0)
        def _():
            pltpu.sync_copy(acc.at[sl], out0.at[sl])

        @pl.when(cid == 1)
        def _():
            pltpu.sync_copy(acc.at[sl], out1.at[sl])

    return k(dst3, zeros_rows)


def _sc_scatter_rows(g, src3, dst3, zeros_rows, d):
    mesh = plsc.VectorSubcoreMesh(**_MESH)

    @functools.partial(
        pl.kernel,
        out_type=(jax.ShapeDtypeStruct((N_PAD, d), jnp.float32),
                  jax.ShapeDtypeStruct((N_PAD, d), jnp.float32)),
        mesh=mesh,
        scratch_types=[
            pltpu.VMEM((NCHUNK, CHUNK), jnp.int32),
            pltpu.VMEM((NCHUNK, CHUNK), jnp.int32),
            pltpu.VMEM((CHUNK, d), jnp.float32),
            pltpu.VMEM((CHUNK, d), jnp.float32),
            pltpu.VMEM_SHARED((N_PAD, d), jnp.float32),
            pltpu.SemaphoreType.DMA,
            pltpu.SemaphoreType.DMA,
        ],
        compiler_params=pltpu.CompilerParams(use_tc_tiling_on_sc=False),
    )
    def k(g_hbm, src_hbm, dst_hbm, z_hbm, out0, out1,
          sidx, didx, rows_a, rows_b, acc, sem_a, sem_b):
        cid = lax.axis_index("c")
        sid = lax.axis_index("s")
        wid = _worker_id()
        pltpu.sync_copy(src_hbm.at[wid], sidx)
        pltpu.sync_copy(dst_hbm.at[wid], didx)
        pltpu.sync_copy(z_hbm.at[sid], acc.at[pl.ds(sid * RPT, RPT)])
        plsc.subcore_barrier()

        def gather_start(ci, rows, sem):
            pltpu.async_copy(g_hbm.at[sidx.at[ci]], rows, sem)

        def gather_wait(ci, rows, sem):
            pltpu.make_async_copy(g_hbm.at[sidx.at[ci]], rows, sem).wait()

        def scatter(ci, rows):
            pltpu.sync_copy(rows, acc.at[didx.at[ci]], add=True)

        gather_start(0, rows_a, sem_a)

        def body(i, carry):
            a = 2 * i
            b = a + 1
            gather_start(b, rows_b, sem_b)
            gather_wait(a, rows_a, sem_a)
            scatter(a, rows_a)
            gather_start(a + 2, rows_a, sem_a)
            gather_wait(b, rows_b, sem_b)
            scatter(b, rows_b)
            return carry

        lax.fori_loop(0, (NCHUNK - 1) // 2, body, 0)
        last = NCHUNK - 1
        gather_wait(last, rows_a, sem_a)
        scatter(last, rows_a)
        plsc.subcore_barrier()
        sl = pl.ds(sid * RPT, RPT)

        @pl.when(cid == 0)
        def _():
            pltpu.sync_copy(acc.at[sl], out0.at[sl])

        @pl.when(cid == 1)
        def _():
            pltpu.sync_copy(acc.at[sl], out1.at[sl])

    return k(g, src3, dst3, zeros_rows)


BM = 2000


def _dis_from(d0, d1):
    return lax.rsqrt(d0 + d1 + 1.0)


def _tc_g1(x, w1, d0, d1):
    def body(x_ref, w_ref, d0_ref, d1_ref, o_ref):
        dis = _dis_from(d0_ref[...], d1_ref[...])
        h = jnp.dot(x_ref[...], w_ref[...], preferred_element_type=jnp.float32)
        o_ref[...] = h * dis

    return pl.pallas_call(
        body,
        grid=(N // BM,),
        in_specs=[
            pl.BlockSpec((BM, F_IN), lambda i: (i, 0)),
            pl.BlockSpec((F_IN, H), lambda i: (0, 0)),
            pl.BlockSpec((BM, 1), lambda i: (i, 0)),
            pl.BlockSpec((BM, 1), lambda i: (i, 0)),
        ],
        out_specs=pl.BlockSpec((BM, H), lambda i: (i, 0)),
        out_shape=jax.ShapeDtypeStruct((N, H), jnp.float32),
    )(x, w1, d0, d1)


def _tc_g2(g1, p0, p1, d0, d1, b1, w2p):
    def body(g_ref, p0_ref, p1_ref, d0_ref, d1_ref, b_ref, w_ref, o_ref):
        dis = _dis_from(d0_ref[...], d1_ref[...])
        z = dis * (p0_ref[...] + p1_ref[...] + g_ref[...]) + b_ref[...]
        z = jnp.maximum(z, 0.0)
        o_ref[...] = jnp.dot(z, w_ref[...], preferred_element_type=jnp.float32) * dis

    return pl.pallas_call(
        body,
        grid=(N // BM,),
        in_specs=[
            pl.BlockSpec((BM, H), lambda i: (i, 0)),
            pl.BlockSpec((BM, H), lambda i: (i, 0)),
            pl.BlockSpec((BM, H), lambda i: (i, 0)),
            pl.BlockSpec((BM, 1), lambda i: (i, 0)),
            pl.BlockSpec((BM, 1), lambda i: (i, 0)),
            pl.BlockSpec((1, H), lambda i: (0, 0)),
            pl.BlockSpec((H, C_PAD), lambda i: (0, 0)),
        ],
        out_specs=pl.BlockSpec((BM, C_PAD), lambda i: (i, 0)),
        out_shape=jax.ShapeDtypeStruct((N, C_PAD), jnp.float32),
    )(g1, p0, p1, d0, d1, b1, w2p)


def _tc_out(g2, q0, q1, d0, d1, b2p):
    def body(g_ref, q0_ref, q1_ref, d0_ref, d1_ref, b_ref, o_ref):
        dis = _dis_from(d0_ref[...], d1_ref[...])
        res = dis * (q0_ref[...] + q1_ref[...] + g_ref[...]) + b_ref[...]
        o_ref[...] = res[:, :C]

    return pl.pallas_call(
        body,
        grid=(N // BM,),
        in_specs=[
            pl.BlockSpec((BM, C_PAD), lambda i: (i, 0)),
            pl.BlockSpec((BM, C_PAD), lambda i: (i, 0)),
            pl.BlockSpec((BM, C_PAD), lambda i: (i, 0)),
            pl.BlockSpec((BM, 1), lambda i: (i, 0)),
            pl.BlockSpec((BM, 1), lambda i: (i, 0)),
            pl.BlockSpec((1, C_PAD), lambda i: (0, 0)),
        ],
        out_specs=pl.BlockSpec((BM, C), lambda i: (i, 0)),
        out_shape=jax.ShapeDtypeStruct((N, C), jnp.float32),
    )(g2, q0, q1, d0, d1, b2p)


def kernel(x, edge_index, W1, b1, W2, b2):
    src3 = edge_index[0].reshape(NW, NCHUNK, CHUNK)
    dst3 = edge_index[1].reshape(NW, NCHUNK, CHUNK)
    w2p = jnp.pad(W2, ((0, 0), (0, C_PAD - C)))
    b1r = b1.reshape(1, H)
    b2p = jnp.pad(b2, (0, C_PAD - C)).reshape(1, C_PAD)
    z1 = jnp.zeros((NS, RPT), jnp.float32)
    zh = jnp.zeros((NS, RPT, H), jnp.float32)
    zc = jnp.zeros((NS, RPT, C_PAD), jnp.float32)

    dg0, dg1 = _sc_degree(dst3, z1)
    d0 = dg0.reshape(N_PAD, 1)
    d1 = dg1.reshape(N_PAD, 1)

    g1 = _tc_g1(x, W1, d0, d1)
    p0, p1 = _sc_scatter_rows(g1, src3, dst3, zh, H)
    g2 = _tc_g2(g1, p0, p1, d0, d1, b1r, w2p)
    q0, q1 = _sc_scatter_rows(g2, src3, dst3, zc, C_PAD)
    out = _tc_out(g2, q0, q1, d0, d1, b2p)
    return out


# speedup vs baseline: 44.5048x; 1.2559x over previous
import functools

import jax
import jax.numpy as jnp
from jax import lax
from jax.experimental import pallas as pl
from jax.experimental.pallas import tpu as pltpu
from jax.experimental.pallas import tpu_sc as plsc

N = 10000
E = 320000
F_IN = 128
H = 64
C = 40
C_PAD = 48

NC = 2
NS = 16
NW = NC * NS
N_PAD = 10240
RPT = N_PAD // NS
EPW = E // NW
CHUNK = 80
NCHUNK = EPW // CHUNK

_MESH = dict(core_axis_name="c", subcore_axis_name="s",
             num_cores=NC, num_subcores=NS)


def _worker_id():
    return lax.axis_index("s") * NC + lax.axis_index("c")


def _sc_degree(dst3, zeros_rows):
    mesh = plsc.VectorSubcoreMesh(**_MESH)

    @functools.partial(
        pl.kernel,
        out_type=(jax.ShapeDtypeStruct((N_PAD,), jnp.float32),
                  jax.ShapeDtypeStruct((N_PAD,), jnp.float32)),
        mesh=mesh,
        scratch_types=[
            pltpu.VMEM((NCHUNK, CHUNK), jnp.int32),
            pltpu.VMEM((CHUNK,), jnp.float32),
            pltpu.VMEM_SHARED((N_PAD,), jnp.float32),
            pltpu.SemaphoreType.DMA,
        ],
    )
    def k(dst_hbm, z_hbm, out0, out1, eidx, ones_v, acc, sem):
        cid = lax.axis_index("c")
        sid = lax.axis_index("s")
        wid = _worker_id()
        for i in range(CHUNK // 16):
            ones_v[pl.ds(16 * i, 16)] = jnp.full((16,), 1.0, jnp.float32)
        pltpu.sync_copy(dst_hbm.at[wid], eidx)
        pltpu.sync_copy(z_hbm.at[sid], acc.at[pl.ds(sid * RPT, RPT)])
        plsc.subcore_barrier()

        LAG = 8

        def body(ci, carry):
            pltpu.async_copy(ones_v, acc.at[eidx.at[ci]], sem, add=True)

            @pl.when(ci >= LAG)
            def _():
                pltpu.make_async_copy(ones_v, acc.at[eidx.at[ci - LAG]],
                                      sem).wait()

            return carry

        lax.fori_loop(0, NCHUNK, body, 0)

        def drain(ci, carry):
            pltpu.make_async_copy(ones_v, acc.at[eidx.at[ci]], sem).wait()
            return carry

        lax.fori_loop(NCHUNK - LAG, NCHUNK, drain, 0)
        plsc.subcore_barrier()
        sl = pl.ds(sid * RPT, RPT)

        @pl.when(cid == 0)
        def _():
            pltpu.sync_copy(acc.at[sl], out0.at[sl])

        @pl.when(cid == 1)
        def _():
            pltpu.sync_copy(acc.at[sl], out1.at[sl])

    return k(dst3, zeros_rows)


def _sc_scatter_rows(g, src3, dst3, zeros_rows, d):
    mesh = plsc.VectorSubcoreMesh(**_MESH)

    @functools.partial(
        pl.kernel,
        out_type=(jax.ShapeDtypeStruct((N_PAD, d), jnp.float32),
                  jax.ShapeDtypeStruct((N_PAD, d), jnp.float32)),
        mesh=mesh,
        scratch_types=[
            pltpu.VMEM((NCHUNK, CHUNK), jnp.int32),
            pltpu.VMEM((NCHUNK, CHUNK), jnp.int32),
            pltpu.VMEM((CHUNK, d), jnp.float32),
            pltpu.VMEM((CHUNK, d), jnp.float32),
            pltpu.VMEM((CHUNK, d), jnp.float32),
            pltpu.VMEM((CHUNK, d), jnp.float32),
            pltpu.VMEM_SHARED((N_PAD, d), jnp.float32),
            pltpu.SemaphoreType.DMA,
            pltpu.SemaphoreType.DMA,
            pltpu.SemaphoreType.DMA,
            pltpu.SemaphoreType.DMA,
        ],
        compiler_params=pltpu.CompilerParams(use_tc_tiling_on_sc=False),
    )
    def k(g_hbm, src_hbm, dst_hbm, z_hbm, out0, out1,
          sidx, didx, r0, r1, r2, r3, acc, s0, s1, s2, s3):
        cid = lax.axis_index("c")
        sid = lax.axis_index("s")
        wid = _worker_id()
        bufs = (r0, r1, r2, r3)
        sems = (s0, s1, s2, s3)
        pltpu.sync_copy(src_hbm.at[wid], sidx)
        pltpu.sync_copy(dst_hbm.at[wid], didx)
        pltpu.sync_copy(z_hbm.at[sid], acc.at[pl.ds(sid * RPT, RPT)])
        plsc.subcore_barrier()

        def gs(ci, b):
            pltpu.async_copy(g_hbm.at[sidx.at[ci]], bufs[b], sems[b])

        def gw(ci, b):
            pltpu.make_async_copy(g_hbm.at[sidx.at[ci]], bufs[b],
                                  sems[b]).wait()

        def ss(ci, b):
            pltpu.async_copy(bufs[b], acc.at[didx.at[ci]], sems[b], add=True)

        def sw(ci, b):
            pltpu.make_async_copy(bufs[b], acc.at[didx.at[ci]],
                                  sems[b]).wait()

        gs(0, 0)
        gs(1, 1)
        gs(2, 2)
        gw(0, 0)
        ss(0, 0)
        gs(3, 3)
        gw(1, 1)
        ss(1, 1)

        def step(c, b):
            sw(c - 2, (b + 2) % 4)
            gs(c + 2, (b + 2) % 4)
            gw(c, b)
            ss(c, b)

        def body(i, carry):
            base = 4 * i + 2
            for j in range(4):
                step(base + j, (2 + j) % 4)
            return carry

        lax.fori_loop(0, 30, body, 0)
        step(122, 2)
        sw(121, 1)
        gw(123, 3)
        ss(123, 3)
        gw(124, 0)
        ss(124, 0)
        sw(122, 2)
        sw(123, 3)
        sw(124, 0)
        plsc.subcore_barrier()
        sl = pl.ds(sid * RPT, RPT)

        @pl.when(cid == 0)
        def _():
            pltpu.sync_copy(acc.at[sl], out0.at[sl])

        @pl.when(cid == 1)
        def _():
            pltpu.sync_copy(acc.at[sl], out1.at[sl])

    return k(g, src3, dst3, zeros_rows)


BM = 2000


def _dis_from(d0, d1):
    return lax.rsqrt(d0 + d1 + 1.0)


def _tc_g1(x, w1, d0, d1):
    def body(x_ref, w_ref, d0_ref, d1_ref, o_ref):
        dis = _dis_from(d0_ref[...], d1_ref[...])
        h = jnp.dot(x_ref[...], w_ref[...], preferred_element_type=jnp.float32)
        o_ref[...] = h * dis

    return pl.pallas_call(
        body,
        grid=(N // BM,),
        in_specs=[
            pl.BlockSpec((BM, F_IN), lambda i: (i, 0)),
            pl.BlockSpec((F_IN, H), lambda i: (0, 0)),
            pl.BlockSpec((BM, 1), lambda i: (i, 0)),
            pl.BlockSpec((BM, 1), lambda i: (i, 0)),
        ],
        out_specs=pl.BlockSpec((BM, H), lambda i: (i, 0)),
        out_shape=jax.ShapeDtypeStruct((N, H), jnp.float32),
    )(x, w1, d0, d1)


def _tc_g2(g1, p0, p1, d0, d1, b1, w2p):
    def body(g_ref, p0_ref, p1_ref, d0_ref, d1_ref, b_ref, w_ref, o_ref):
        dis = _dis_from(d0_ref[...], d1_ref[...])
        z = dis * (p0_ref[...] + p1_ref[...] + g_ref[...]) + b_ref[...]
        z = jnp.maximum(z, 0.0)
        o_ref[...] = jnp.dot(z, w_ref[...], preferred_element_type=jnp.float32) * dis

    return pl.pallas_call(
        body,
        grid=(N // BM,),
        in_specs=[
            pl.BlockSpec((BM, H), lambda i: (i, 0)),
            pl.BlockSpec((BM, H), lambda i: (i, 0)),
            pl.BlockSpec((BM, H), lambda i: (i, 0)),
            pl.BlockSpec((BM, 1), lambda i: (i, 0)),
            pl.BlockSpec((BM, 1), lambda i: (i, 0)),
            pl.BlockSpec((1, H), lambda i: (0, 0)),
            pl.BlockSpec((H, C_PAD), lambda i: (0, 0)),
        ],
        out_specs=pl.BlockSpec((BM, C_PAD), lambda i: (i, 0)),
        out_shape=jax.ShapeDtypeStruct((N, C_PAD), jnp.float32),
    )(g1, p0, p1, d0, d1, b1, w2p)


def _tc_out(g2, q0, q1, d0, d1, b2p):
    def body(g_ref, q0_ref, q1_ref, d0_ref, d1_ref, b_ref, o_ref):
        dis = _dis_from(d0_ref[...], d1_ref[...])
        res = dis * (q0_ref[...] + q1_ref[...] + g_ref[...]) + b_ref[...]
        o_ref[...] = res[:, :C]

    return pl.pallas_call(
        body,
        grid=(N // BM,),
        in_specs=[
            pl.BlockSpec((BM, C_PAD), lambda i: (i, 0)),
            pl.BlockSpec((BM, C_PAD), lambda i: (i, 0)),
            pl.BlockSpec((BM, C_PAD), lambda i: (i, 0)),
            pl.BlockSpec((BM, 1), lambda i: (i, 0)),
            pl.BlockSpec((BM, 1), lambda i: (i, 0)),
            pl.BlockSpec((1, C_PAD), lambda i: (0, 0)),
        ],
        out_specs=pl.BlockSpec((BM, C), lambda i: (i, 0)),
        out_shape=jax.ShapeDtypeStruct((N, C), jnp.float32),
    )(g2, q0, q1, d0, d1, b2p)


def kernel(x, edge_index, W1, b1, W2, b2):
    src3 = edge_index[0].reshape(NW, NCHUNK, CHUNK)
    dst3 = edge_index[1].reshape(NW, NCHUNK, CHUNK)
    w2p = jnp.pad(W2, ((0, 0), (0, C_PAD - C)))
    b1r = b1.reshape(1, H)
    b2p = jnp.pad(b2, (0, C_PAD - C)).reshape(1, C_PAD)
    z1 = jnp.zeros((NS, RPT), jnp.float32)
    zh = jnp.zeros((NS, RPT, H), jnp.float32)
    zc = jnp.zeros((NS, RPT, C_PAD), jnp.float32)

    dg0, dg1 = _sc_degree(dst3, z1)
    d0 = dg0.reshape(N_PAD, 1)
    d1 = dg1.reshape(N_PAD, 1)

    g1 = _tc_g1(x, W1, d0, d1)
    p0, p1 = _sc_scatter_rows(g1, src3, dst3, zh, H)
    g2 = _tc_g2(g1, p0, p1, d0, d1, b1r, w2p)
    q0, q1 = _sc_scatter_rows(g2, src3, dst3, zc, C_PAD)
    out = _tc_out(g2, q0, q1, d0, d1, b2p)
    return out


# speedup vs baseline: 44.5610x; 1.0013x over previous
import functools

import jax
import jax.numpy as jnp
from jax import lax
from jax.experimental import pallas as pl
from jax.experimental.pallas import tpu as pltpu
from jax.experimental.pallas import tpu_sc as plsc

N = 10000
E = 320000
F_IN = 128
H = 64
C = 40
C_PAD = 48

NC = 2
NS = 16
NW = NC * NS
N_PAD = 10240
RPT = N_PAD // NS
EPW = E // NW
CHUNK = 80
NCHUNK = EPW // CHUNK

_MESH = dict(core_axis_name="c", subcore_axis_name="s",
             num_cores=NC, num_subcores=NS)


def _worker_id():
    return lax.axis_index("s") * NC + lax.axis_index("c")


def _sc_degree(dst3, zeros_rows):
    mesh = plsc.VectorSubcoreMesh(**_MESH)

    @functools.partial(
        pl.kernel,
        out_type=(jax.ShapeDtypeStruct((N_PAD,), jnp.float32),
                  jax.ShapeDtypeStruct((N_PAD,), jnp.float32)),
        mesh=mesh,
        scratch_types=[
            pltpu.VMEM((NCHUNK, CHUNK), jnp.int32),
            pltpu.VMEM((CHUNK,), jnp.float32),
            pltpu.VMEM_SHARED((N_PAD,), jnp.float32),
            pltpu.SemaphoreType.DMA,
        ],
    )
    def k(dst_hbm, z_hbm, out0, out1, eidx, ones_v, acc, sem):
        cid = lax.axis_index("c")
        sid = lax.axis_index("s")
        wid = _worker_id()
        for i in range(CHUNK // 16):
            ones_v[pl.ds(16 * i, 16)] = jnp.full((16,), 1.0, jnp.float32)
        pltpu.sync_copy(dst_hbm.at[wid], eidx)
        pltpu.sync_copy(z_hbm.at[sid], acc.at[pl.ds(sid * RPT, RPT)])
        plsc.subcore_barrier()

        LAG = 8

        def body(ci, carry):
            pltpu.async_copy(ones_v, acc.at[eidx.at[ci]], sem, add=True)

            @pl.when(ci >= LAG)
            def _():
                pltpu.make_async_copy(ones_v, acc.at[eidx.at[ci - LAG]],
                                      sem).wait()

            return carry

        lax.fori_loop(0, NCHUNK, body, 0)

        def drain(ci, carry):
            pltpu.make_async_copy(ones_v, acc.at[eidx.at[ci]], sem).wait()
            return carry

        lax.fori_loop(NCHUNK - LAG, NCHUNK, drain, 0)
        plsc.subcore_barrier()
        sl = pl.ds(sid * RPT, RPT)

        @pl.when(cid == 0)
        def _():
            pltpu.sync_copy(acc.at[sl], out0.at[sl])

        @pl.when(cid == 1)
        def _():
            pltpu.sync_copy(acc.at[sl], out1.at[sl])

    return k(dst3, zeros_rows)


def _sc_scatter_rows(g, src3, dst3, zeros_rows, d):
    mesh = plsc.VectorSubcoreMesh(**_MESH)

    @functools.partial(
        pl.kernel,
        out_type=(jax.ShapeDtypeStruct((N_PAD, d), jnp.float32),
                  jax.ShapeDtypeStruct((N_PAD, d), jnp.float32)),
        mesh=mesh,
        scratch_types=[
            pltpu.VMEM((NCHUNK, CHUNK), jnp.int32),
            pltpu.VMEM((NCHUNK, CHUNK), jnp.int32),
            pltpu.VMEM((CHUNK, d), jnp.float32),
            pltpu.VMEM((CHUNK, d), jnp.float32),
            pltpu.VMEM((CHUNK, d), jnp.float32),
            pltpu.VMEM((CHUNK, d), jnp.float32),
            pltpu.VMEM_SHARED((N_PAD, d), jnp.float32),
            pltpu.SemaphoreType.DMA,
            pltpu.SemaphoreType.DMA,
            pltpu.SemaphoreType.DMA,
            pltpu.SemaphoreType.DMA,
        ],
        compiler_params=pltpu.CompilerParams(use_tc_tiling_on_sc=False),
    )
    def k(g_hbm, src_hbm, dst_hbm, z_hbm, out0, out1,
          sidx, didx, r0, r1, r2, r3, acc, s0, s1, s2, s3):
        cid = lax.axis_index("c")
        sid = lax.axis_index("s")
        wid = _worker_id()
        bufs = (r0, r1, r2, r3)
        sems = (s0, s1, s2, s3)
        pltpu.sync_copy(src_hbm.at[wid], sidx)
        pltpu.sync_copy(dst_hbm.at[wid], didx)
        pltpu.sync_copy(z_hbm.at[sid], acc.at[pl.ds(sid * RPT, RPT)])
        plsc.subcore_barrier()

        def gs(ci, b):
            pltpu.async_copy(g_hbm.at[sidx.at[ci]], bufs[b], sems[b])

        def gw(ci, b):
            pltpu.make_async_copy(g_hbm.at[sidx.at[ci]], bufs[b],
                                  sems[b]).wait()

        def ss(ci, b):
            pltpu.async_copy(bufs[b], acc.at[didx.at[ci]], sems[b], add=True)

        def sw(ci, b):
            pltpu.make_async_copy(bufs[b], acc.at[didx.at[ci]],
                                  sems[b]).wait()

        gs(0, 0)
        gs(1, 1)
        gs(2, 2)
        gw(0, 0)
        ss(0, 0)
        gs(3, 3)
        gw(1, 1)
        ss(1, 1)

        def step(c, b):
            sw(c - 2, (b + 2) % 4)
            gs(c + 2, (b + 2) % 4)
            gw(c, b)
            ss(c, b)

        def body(i, carry):
            base = 4 * i + 2
            for j in range(4):
                step(base + j, (2 + j) % 4)
            return carry

        lax.fori_loop(0, 30, body, 0)
        step(122, 2)
        sw(121, 1)
        gw(123, 3)
        ss(123, 3)
        gw(124, 0)
        ss(124, 0)
        sw(122, 2)
        sw(123, 3)
        sw(124, 0)
        plsc.subcore_barrier()
        sl = pl.ds(sid * RPT, RPT)

        @pl.when(cid == 0)
        def _():
            pltpu.sync_copy(acc.at[sl], out0.at[sl])

        @pl.when(cid == 1)
        def _():
            pltpu.sync_copy(acc.at[sl], out1.at[sl])

    return k(g, src3, dst3, zeros_rows)


BM = 5000


def _dis_from(d0, d1):
    return lax.rsqrt(d0 + d1 + 1.0)


def _tc_g1(x, w1, d0, d1):
    def body(x_ref, w_ref, d0_ref, d1_ref, o_ref):
        dis = _dis_from(d0_ref[...], d1_ref[...])
        h = jnp.dot(x_ref[...], w_ref[...], preferred_element_type=jnp.float32)
        o_ref[...] = h * dis

    return pl.pallas_call(
        body,
        grid=(N // BM,),
        in_specs=[
            pl.BlockSpec((BM, F_IN), lambda i: (i, 0)),
            pl.BlockSpec((F_IN, H), lambda i: (0, 0)),
            pl.BlockSpec((BM, 1), lambda i: (i, 0)),
            pl.BlockSpec((BM, 1), lambda i: (i, 0)),
        ],
        out_specs=pl.BlockSpec((BM, H), lambda i: (i, 0)),
        out_shape=jax.ShapeDtypeStruct((N, H), jnp.float32),
    )(x, w1, d0, d1)


def _tc_g2(g1, p0, p1, d0, d1, b1, w2p):
    def body(g_ref, p0_ref, p1_ref, d0_ref, d1_ref, b_ref, w_ref, o_ref):
        dis = _dis_from(d0_ref[...], d1_ref[...])
        z = dis * (p0_ref[...] + p1_ref[...] + g_ref[...]) + b_ref[...]
        z = jnp.maximum(z, 0.0)
        o_ref[...] = jnp.dot(z, w_ref[...], preferred_element_type=jnp.float32) * dis

    return pl.pallas_call(
        body,
        grid=(N // BM,),
        in_specs=[
            pl.BlockSpec((BM, H), lambda i: (i, 0)),
            pl.BlockSpec((BM, H), lambda i: (i, 0)),
            pl.BlockSpec((BM, H), lambda i: (i, 0)),
            pl.BlockSpec((BM, 1), lambda i: (i, 0)),
            pl.BlockSpec((BM, 1), lambda i: (i, 0)),
            pl.BlockSpec((1, H), lambda i: (0, 0)),
            pl.BlockSpec((H, C_PAD), lambda i: (0, 0)),
        ],
        out_specs=pl.BlockSpec((BM, C_PAD), lambda i: (i, 0)),
        out_shape=jax.ShapeDtypeStruct((N, C_PAD), jnp.float32),
    )(g1, p0, p1, d0, d1, b1, w2p)


def _tc_out(g2, q0, q1, d0, d1, b2p):
    def body(g_ref, q0_ref, q1_ref, d0_ref, d1_ref, b_ref, o_ref):
        dis = _dis_from(d0_ref[...], d1_ref[...])
        res = dis * (q0_ref[...] + q1_ref[...] + g_ref[...]) + b_ref[...]
        o_ref[...] = res[:, :C]

    return pl.pallas_call(
        body,
        grid=(N // BM,),
        in_specs=[
            pl.BlockSpec((BM, C_PAD), lambda i: (i, 0)),
            pl.BlockSpec((BM, C_PAD), lambda i: (i, 0)),
            pl.BlockSpec((BM, C_PAD), lambda i: (i, 0)),
            pl.BlockSpec((BM, 1), lambda i: (i, 0)),
            pl.BlockSpec((BM, 1), lambda i: (i, 0)),
            pl.BlockSpec((1, C_PAD), lambda i: (0, 0)),
        ],
        out_specs=pl.BlockSpec((BM, C), lambda i: (i, 0)),
        out_shape=jax.ShapeDtypeStruct((N, C), jnp.float32),
    )(g2, q0, q1, d0, d1, b2p)


def kernel(x, edge_index, W1, b1, W2, b2):
    src3 = edge_index[0].reshape(NW, NCHUNK, CHUNK)
    dst3 = edge_index[1].reshape(NW, NCHUNK, CHUNK)
    w2p = jnp.pad(W2, ((0, 0), (0, C_PAD - C)))
    b1r = b1.reshape(1, H)
    b2p = jnp.pad(b2, (0, C_PAD - C)).reshape(1, C_PAD)
    z1 = jnp.zeros((NS, RPT), jnp.float32)
    zh = jnp.zeros((NS, RPT, H), jnp.float32)
    zc = jnp.zeros((NS, RPT, C_PAD), jnp.float32)

    dg0, dg1 = _sc_degree(dst3, z1)
    d0 = dg0.reshape(N_PAD, 1)
    d1 = dg1.reshape(N_PAD, 1)

    g1 = _tc_g1(x, W1, d0, d1)
    p0, p1 = _sc_scatter_rows(g1, src3, dst3, zh, H)
    g2 = _tc_g2(g1, p0, p1, d0, d1, b1r, w2p)
    q0, q1 = _sc_scatter_rows(g2, src3, dst3, zc, C_PAD)
    out = _tc_out(g2, q0, q1, d0, d1, b2p)
    return out


# speedup vs baseline: 44.5624x; 1.0000x over previous
import functools

import jax
import jax.numpy as jnp
from jax import lax
from jax.experimental import pallas as pl
from jax.experimental.pallas import tpu as pltpu
from jax.experimental.pallas import tpu_sc as plsc

N = 10000
E = 320000
F_IN = 128
H = 64
C = 40
C_PAD = 48

NC = 2
NS = 16
NW = NC * NS
N_PAD = 10240
RPT = N_PAD // NS
EPW = E // NW
CHUNK = 80
NCHUNK = EPW // CHUNK

_MESH = dict(core_axis_name="c", subcore_axis_name="s",
             num_cores=NC, num_subcores=NS)


def _worker_id():
    return lax.axis_index("s") * NC + lax.axis_index("c")


def _sc_degree(dst3, zeros_rows):
    mesh = plsc.VectorSubcoreMesh(**_MESH)

    @functools.partial(
        pl.kernel,
        out_type=(jax.ShapeDtypeStruct((N_PAD,), jnp.float32),
                  jax.ShapeDtypeStruct((N_PAD,), jnp.float32)),
        mesh=mesh,
        scratch_types=[
            pltpu.VMEM((NCHUNK, CHUNK), jnp.int32),
            pltpu.VMEM((CHUNK,), jnp.float32),
            pltpu.VMEM_SHARED((N_PAD,), jnp.float32),
            pltpu.SemaphoreType.DMA,
        ],
    )
    def k(dst_hbm, z_hbm, out0, out1, eidx, ones_v, acc, sem):
        cid = lax.axis_index("c")
        sid = lax.axis_index("s")
        wid = _worker_id()
        for i in range(CHUNK // 16):
            ones_v[pl.ds(16 * i, 16)] = jnp.full((16,), 1.0, jnp.float32)
        pltpu.sync_copy(dst_hbm.at[wid], eidx)
        pltpu.sync_copy(z_hbm.at[sid], acc.at[pl.ds(sid * RPT, RPT)])
        plsc.subcore_barrier()

        LAG = 8

        def body(ci, carry):
            pltpu.async_copy(ones_v, acc.at[eidx.at[ci]], sem, add=True)

            @pl.when(ci >= LAG)
            def _():
                pltpu.make_async_copy(ones_v, acc.at[eidx.at[ci - LAG]],
                                      sem).wait()

            return carry

        lax.fori_loop(0, NCHUNK, body, 0)

        def drain(ci, carry):
            pltpu.make_async_copy(ones_v, acc.at[eidx.at[ci]], sem).wait()
            return carry

        lax.fori_loop(NCHUNK - LAG, NCHUNK, drain, 0)
        plsc.subcore_barrier()
        sl = pl.ds(sid * RPT, RPT)

        @pl.when(cid == 0)
        def _():
            pltpu.sync_copy(acc.at[sl], out0.at[sl])

        @pl.when(cid == 1)
        def _():
            pltpu.sync_copy(acc.at[sl], out1.at[sl])

    return k(dst3, zeros_rows)


def _sc_scatter_rows(g, src3, dst3, zeros_rows, d):
    mesh = plsc.VectorSubcoreMesh(**_MESH)

    @functools.partial(
        pl.kernel,
        out_type=(jax.ShapeDtypeStruct((N_PAD, d), jnp.float32),
                  jax.ShapeDtypeStruct((N_PAD, d), jnp.float32)),
        mesh=mesh,
        scratch_types=[
            pltpu.VMEM((NCHUNK, CHUNK), jnp.int32),
            pltpu.VMEM((NCHUNK, CHUNK), jnp.int32),
        ] + [pltpu.VMEM((CHUNK, d), jnp.float32)] * 4 + [
            pltpu.VMEM_SHARED((N_PAD, d), jnp.float32),
        ] + [pltpu.SemaphoreType.DMA] * 4,
        compiler_params=pltpu.CompilerParams(use_tc_tiling_on_sc=False),
    )
    def k(g_hbm, src_hbm, dst_hbm, z_hbm, out0, out1,
          sidx, didx, r0, r1, r2, r3, acc, s0, s1, s2, s3):
        cid = lax.axis_index("c")
        sid = lax.axis_index("s")
        wid = _worker_id()
        bufs = (r0, r1, r2, r3)
        sems = (s0, s1, s2, s3)
        pltpu.sync_copy(src_hbm.at[wid], sidx)
        pltpu.sync_copy(dst_hbm.at[wid], didx)
        pltpu.sync_copy(z_hbm.at[sid], acc.at[pl.ds(sid * RPT, RPT)])
        plsc.subcore_barrier()

        def gs(ci, b):
            pltpu.async_copy(g_hbm.at[sidx.at[ci]], bufs[b], sems[b])

        def gw(ci, b):
            pltpu.make_async_copy(g_hbm.at[sidx.at[ci]], bufs[b],
                                  sems[b]).wait()

        def ss(ci, b):
            pltpu.async_copy(bufs[b], acc.at[didx.at[ci]], sems[b], add=True)

        def sw(ci, b):
            pltpu.make_async_copy(bufs[b], acc.at[didx.at[ci]],
                                  sems[b]).wait()

        gs(0, 0)
        gs(1, 1)
        gs(2, 2)
        gw(0, 0)
        ss(0, 0)
        gs(3, 3)
        gw(1, 1)
        ss(1, 1)

        def step(c, b):
            sw(c - 2, (b + 2) % 4)
            gs(c + 2, (b + 2) % 4)
            gw(c, b)
            ss(c, b)

        def body(i, carry):
            base = 4 * i + 2
            for j in range(4):
                step(base + j, (2 + j) % 4)
            return carry

        lax.fori_loop(0, 30, body, 0)
        step(122, 2)
        sw(121, 1)
        gw(123, 3)
        ss(123, 3)
        gw(124, 0)
        ss(124, 0)
        sw(122, 2)
        sw(123, 3)
        sw(124, 0)
        plsc.subcore_barrier()
        sl = pl.ds(sid * RPT, RPT)

        @pl.when(cid == 0)
        def _():
            pltpu.sync_copy(acc.at[sl], out0.at[sl])

        @pl.when(cid == 1)
        def _():
            pltpu.sync_copy(acc.at[sl], out1.at[sl])

    return k(g, src3, dst3, zeros_rows)


BM = 5000


def _dis_from(d0, d1):
    return lax.rsqrt(d0 + d1 + 1.0)


def _tc_g1(x, w1, d0, d1):
    def body(x_ref, w_ref, d0_ref, d1_ref, o_ref):
        dis = _dis_from(d0_ref[...], d1_ref[...])
        h = jnp.dot(x_ref[...], w_ref[...], preferred_element_type=jnp.float32)
        o_ref[...] = h * dis

    return pl.pallas_call(
        body,
        grid=(N // BM,),
        in_specs=[
            pl.BlockSpec((BM, F_IN), lambda i: (i, 0)),
            pl.BlockSpec((F_IN, H), lambda i: (0, 0)),
            pl.BlockSpec((BM, 1), lambda i: (i, 0)),
            pl.BlockSpec((BM, 1), lambda i: (i, 0)),
        ],
        out_specs=pl.BlockSpec((BM, H), lambda i: (i, 0)),
        out_shape=jax.ShapeDtypeStruct((N, H), jnp.float32),
    )(x, w1, d0, d1)


def _tc_g2(g1, p0, p1, d0, d1, b1, w2p):
    def body(g_ref, p0_ref, p1_ref, d0_ref, d1_ref, b_ref, w_ref, o_ref):
        dis = _dis_from(d0_ref[...], d1_ref[...])
        z = dis * (p0_ref[...] + p1_ref[...] + g_ref[...]) + b_ref[...]
        z = jnp.maximum(z, 0.0)
        o_ref[...] = jnp.dot(z, w_ref[...], preferred_element_type=jnp.float32) * dis

    return pl.pallas_call(
        body,
        grid=(N // BM,),
        in_specs=[
            pl.BlockSpec((BM, H), lambda i: (i, 0)),
            pl.BlockSpec((BM, H), lambda i: (i, 0)),
            pl.BlockSpec((BM, H), lambda i: (i, 0)),
            pl.BlockSpec((BM, 1), lambda i: (i, 0)),
            pl.BlockSpec((BM, 1), lambda i: (i, 0)),
            pl.BlockSpec((1, H), lambda i: (0, 0)),
            pl.BlockSpec((H, C_PAD), lambda i: (0, 0)),
        ],
        out_specs=pl.BlockSpec((BM, C_PAD), lambda i: (i, 0)),
        out_shape=jax.ShapeDtypeStruct((N, C_PAD), jnp.float32),
    )(g1, p0, p1, d0, d1, b1, w2p)


def _tc_out(g2, q0, q1, d0, d1, b2p):
    def body(g_ref, q0_ref, q1_ref, d0_ref, d1_ref, b_ref, o_ref):
        dis = _dis_from(d0_ref[...], d1_ref[...])
        res = dis * (q0_ref[...] + q1_ref[...] + g_ref[...]) + b_ref[...]
        o_ref[...] = res[:, :C]

    return pl.pallas_call(
        body,
        grid=(N // BM,),
        in_specs=[
            pl.BlockSpec((BM, C_PAD), lambda i: (i, 0)),
            pl.BlockSpec((BM, C_PAD), lambda i: (i, 0)),
            pl.BlockSpec((BM, C_PAD), lambda i: (i, 0)),
            pl.BlockSpec((BM, 1), lambda i: (i, 0)),
            pl.BlockSpec((BM, 1), lambda i: (i, 0)),
            pl.BlockSpec((1, C_PAD), lambda i: (0, 0)),
        ],
        out_specs=pl.BlockSpec((BM, C), lambda i: (i, 0)),
        out_shape=jax.ShapeDtypeStruct((N, C), jnp.float32),
    )(g2, q0, q1, d0, d1, b2p)


def kernel(x, edge_index, W1, b1, W2, b2):
    src3 = edge_index[0].reshape(NW, NCHUNK, CHUNK)
    dst3 = edge_index[1].reshape(NW, NCHUNK, CHUNK)
    w2p = jnp.pad(W2, ((0, 0), (0, C_PAD - C)))
    b1r = b1.reshape(1, H)
    b2p = jnp.pad(b2, (0, C_PAD - C)).reshape(1, C_PAD)
    z1 = jnp.zeros((NS, RPT), jnp.float32)
    zh = jnp.zeros((NS, RPT, H), jnp.float32)
    zc = jnp.zeros((NS, RPT, C_PAD), jnp.float32)

    dg0, dg1 = _sc_degree(dst3, z1)
    d0 = dg0.reshape(N_PAD, 1)
    d1 = dg1.reshape(N_PAD, 1)

    g1 = _tc_g1(x, W1, d0, d1)
    p0, p1 = _sc_scatter_rows(g1, src3, dst3, zh, H)
    g2 = _tc_g2(g1, p0, p1, d0, d1, b1r, w2p)
    q0, q1 = _sc_scatter_rows(g2, src3, dst3, zc, C_PAD)
    out = _tc_out(g2, q0, q1, d0, d1, b2p)
    return out
